# baseline (speedup 1.0000x reference)
import dataclasses
import functools

import jax
import jax.numpy as jnp
from jax import lax
from jax.experimental import pallas as pl
from jax.experimental.pallas import tpu as pltpu
from jax.experimental.pallas import tpu_sc as plsc

_B = 16384
_D = 64
_NC = 2
_NS = 16
_NW = _NC * _NS
_CHUNK = 128
_BPW = _B // _NW
_CPW = _BPW // _CHUNK
_LANES = 16


def _make_sc_dot():
    mesh = plsc.VectorSubcoreMesh(core_axis_name="c", subcore_axis_name="s")
    cp = pltpu.CompilerParams()
    if "needs_layout_passes" in pltpu.CompilerParams.__dataclass_fields__:
        cp = dataclasses.replace(cp, needs_layout_passes=False)
    if "use_tc_tiling_on_sc" in pltpu.CompilerParams.__dataclass_fields__:
        cp = dataclasses.replace(cp, use_tc_tiling_on_sc=False)

    @functools.partial(
        pl.kernel,
        mesh=mesh,
        compiler_params=cp,
        out_type=jax.ShapeDtypeStruct((_B,), jnp.float32),
        scratch_types=[
            pltpu.VMEM((_CPW, _CHUNK), jnp.int32),
            pltpu.VMEM((_CPW, _CHUNK), jnp.int32),
            pltpu.VMEM((_BPW, _D), jnp.float32),
            pltpu.VMEM((_BPW, _D), jnp.float32),
            pltpu.VMEM((_BPW,), jnp.float32),
            pltpu.SemaphoreType.DMA,
        ],
    )
    def sc_dot(uid_hbm, iid_hbm, ut_hbm, it_hbm, out_hbm,
               uidx_v, iidx_v, urows_v, irows_v, out_v, sem):
        wid = lax.axis_index("s") * _NC + lax.axis_index("c")
        base = wid * _BPW

        pltpu.sync_copy(uid_hbm.at[pl.ds(wid * _CPW, _CPW)], uidx_v)
        pltpu.sync_copy(iid_hbm.at[pl.ds(wid * _CPW, _CPW)], iidx_v)

        copies = []
        for j in range(_CPW):
            dst = pl.ds(j * _CHUNK, _CHUNK)
            copies.append(pltpu.async_copy(
                ut_hbm.at[uidx_v.at[j]], urows_v.at[dst], sem))
            copies.append(pltpu.async_copy(
                it_hbm.at[iidx_v.at[j]], irows_v.at[dst], sem))
        for c in copies:
            c.wait()

        @pl.loop(0, _BPW, step=_LANES)
        def _(r0):
            rows = r0 + lax.iota(jnp.int32, _LANES)
            acc = None
            for d in range(_D):
                cols = jnp.full((_LANES,), d, jnp.int32)
                u = plsc.load_gather(urows_v, [rows, cols])
                v = plsc.load_gather(irows_v, [rows, cols])
                prod = u * v
                acc = prod if acc is None else acc + prod
            out_v[pl.ds(r0, _LANES)] = acc

        pltpu.sync_copy(out_v, out_hbm.at[pl.ds(base, _BPW)])

    return sc_dot


_sc_dot = _make_sc_dot()


def kernel(input_user_id, input_item_id, user_table, item_table):
    uid = input_user_id.reshape(_B // _CHUNK, _CHUNK).astype(jnp.int32)
    iid = input_item_id.reshape(_B // _CHUNK, _CHUNK).astype(jnp.int32)
    out = _sc_dot(uid, iid, user_table, item_table)
    return out.reshape(_B, 1)


# speedup vs baseline: 1.6081x; 1.6081x over previous
import dataclasses
import functools

import jax
import jax.numpy as jnp
from jax import lax
from jax.experimental import pallas as pl
from jax.experimental.pallas import tpu as pltpu
from jax.experimental.pallas import tpu_sc as plsc

_B = 16384
_D = 64
_NC = 2
_NS = 16
_NW = _NC * _NS
_BPW = _B // _NW
_ROWCHUNK = 128
_NCHUNK = _BPW // _ROWCHUNK
_LANES = 16
_BUFW = 128


def _make_sc_dot():
    mesh = plsc.VectorSubcoreMesh(core_axis_name="c", subcore_axis_name="s")
    cp = pltpu.CompilerParams()
    if "needs_layout_passes" in pltpu.CompilerParams.__dataclass_fields__:
        cp = dataclasses.replace(cp, needs_layout_passes=False)
    if "use_tc_tiling_on_sc" in pltpu.CompilerParams.__dataclass_fields__:
        cp = dataclasses.replace(cp, use_tc_tiling_on_sc=True)

    @functools.partial(
        pl.kernel,
        mesh=mesh,
        compiler_params=cp,
        out_type=jax.ShapeDtypeStruct((_B,), jnp.float32),
        scratch_types=[
            pltpu.VMEM((_ROWCHUNK, _BUFW), jnp.float32),
            pltpu.VMEM((_ROWCHUNK, _BUFW), jnp.float32),
            pltpu.VMEM((_ROWCHUNK, _BUFW), jnp.float32),
            pltpu.VMEM((_ROWCHUNK, _BUFW), jnp.float32),
            pltpu.VMEM((_BPW,), jnp.float32),
            pltpu.VMEM((2 * _BPW,), jnp.int32),
            pltpu.VMEM((_ROWCHUNK * _D,), jnp.float32),
            pltpu.SemaphoreType.DMA,
            pltpu.SemaphoreType.DMA,
        ],
    )
    def sc_dot(uid_hbm, iid_hbm, ut_hbm, it_hbm, out_hbm,
               ub0, ub1, ib0, ib1, out_v, idx_v, dummy_v,
               sem0, sem1):
        ubufs = (ub0, ub1)
        ibufs = (ib0, ib1)
        sems = (sem0, sem1)
        wid = lax.axis_index("s") * _NC + lax.axis_index("c")
        base = wid * _BPW

        pltpu.sync_copy(uid_hbm.at[pl.ds(base, _BPW)], idx_v.at[pl.ds(0, _BPW)])
        pltpu.sync_copy(iid_hbm.at[pl.ds(base, _BPW)], idx_v.at[pl.ds(_BPW, _BPW)])

        def fire(c):
            ub, ib, sem = ubufs[c % 2], ibufs[c % 2], sems[c % 2]

            @pl.loop(0, _ROWCHUNK, step=_LANES)
            def _(j0):
                i0 = c * _ROWCHUNK + j0
                uvec = idx_v[pl.ds(i0, _LANES)]
                ivec = idx_v[pl.ds(_BPW + i0, _LANES)]
                dcol = pl.ds(0, _D)
                for l in range(_LANES):
                    pltpu.async_copy(ut_hbm.at[uvec[l]], ub.at[j0 + l, dcol], sem)
                    pltpu.async_copy(it_hbm.at[ivec[l]], ib.at[j0 + l, dcol], sem)

        def drain(c):
            sem = sems[c % 2]
            dummy_src = out_hbm.at[pl.ds(0, _ROWCHUNK * _D)]
            pltpu.make_async_copy(dummy_src, dummy_v, sem).wait()
            pltpu.make_async_copy(dummy_src, dummy_v, sem).wait()

        def compute(c):
            ub, ib = ubufs[c % 2], ibufs[c % 2]

            @pl.loop(0, _ROWCHUNK, step=_LANES)
            def _(r0):
                rows = r0 + lax.iota(jnp.int32, _LANES)
                acc = None
                for d in range(_D):
                    cols = jnp.full((_LANES,), d, jnp.int32)
                    u = plsc.load_gather(ub, [rows, cols])
                    v = plsc.load_gather(ib, [rows, cols])
                    prod = u * v
                    acc = prod if acc is None else acc + prod
                out_v[pl.ds(c * _ROWCHUNK + r0, _LANES)] = acc

        fire(0)
        for c in range(_NCHUNK):
            if c + 1 < _NCHUNK:
                fire(c + 1)
            drain(c)
            compute(c)

        pltpu.sync_copy(out_v, out_hbm.at[pl.ds(base, _BPW)])

    return sc_dot


_sc_dot = _make_sc_dot()


def kernel(input_user_id, input_item_id, user_table, item_table):
    uid = input_user_id.reshape(_B).astype(jnp.int32)
    iid = input_item_id.reshape(_B).astype(jnp.int32)
    out = _sc_dot(uid, iid, user_table, item_table)
    return out.reshape(_B, 1)
